# baseline (speedup 1.0000x reference)
import jax
import jax.numpy as jnp
from jax.experimental import pallas as pl
from jax.experimental.pallas import tpu as pltpu

_HEADS = 4
_CHUNK = 2048
_NBUF = 8


def _proj_kernel(x_hbm, w_ref, b_ref, o_hbm, x_buf, y_buf, in_sems, out_sems):
    n_chunks = x_hbm.shape[0] // _CHUNK
    wb = w_ref[...].astype(jnp.bfloat16)
    bias = b_ref[...]

    def in_copy(i, j):
        return pltpu.make_async_copy(
            x_hbm.at[pl.ds(i * _CHUNK, _CHUNK), :], x_buf.at[j], in_sems.at[j]
        )

    def out_copy(i, j):
        return pltpu.make_async_copy(
            y_buf.at[j], o_hbm.at[:, pl.ds(i * _CHUNK, _CHUNK)], out_sems.at[j]
        )

    for j in range(min(_NBUF, n_chunks)):
        in_copy(j, j).start()

    for i in range(n_chunks):
        j = i % _NBUF
        in_copy(i, j).wait()
        if i >= _NBUF:
            out_copy(i - _NBUF, j).wait()
        xb = x_buf[j].astype(jnp.bfloat16)
        y_buf[j] = (
            jax.lax.dot_general(
                wb, xb, (((0,), (1,)), ((), ())),
                preferred_element_type=jnp.float32,
            )
            + bias
        )
        out_copy(i, j).start()
        if i + _NBUF < n_chunks:
            in_copy(i + _NBUF, j).start()

    for i in range(max(0, n_chunks - _NBUF), n_chunks):
        out_copy(i, i % _NBUF).wait()


@jax.jit
def kernel(x, W, b):
    M, K = x.shape
    N = W.shape[1]
    b2 = b.reshape(N, 1)
    yt = pl.pallas_call(
        _proj_kernel,
        in_specs=[
            pl.BlockSpec(memory_space=pltpu.MemorySpace.HBM),
            pl.BlockSpec(memory_space=pltpu.MemorySpace.VMEM),
            pl.BlockSpec(memory_space=pltpu.MemorySpace.VMEM),
        ],
        out_specs=pl.BlockSpec(memory_space=pltpu.MemorySpace.HBM),
        out_shape=jax.ShapeDtypeStruct((N, M), jnp.float32),
        scratch_shapes=[
            pltpu.VMEM((_NBUF, _CHUNK, K), jnp.float32),
            pltpu.VMEM((_NBUF, N, _CHUNK), jnp.float32),
            pltpu.SemaphoreType.DMA((_NBUF,)),
            pltpu.SemaphoreType.DMA((_NBUF,)),
        ],
    )(x, W, b2)
    return yt.reshape(_HEADS, N // _HEADS, M).transpose(2, 0, 1)


# speedup vs baseline: 1.0358x; 1.0358x over previous
import jax
import jax.numpy as jnp
from jax.experimental import pallas as pl
from jax.experimental.pallas import tpu as pltpu

_HEADS = 4
_BODY = 4096
_NBUF = 6


def _schedule(m):
    ramp = [512, 512, 1024, 2048]
    fill = m - 2 * sum(ramp)
    sizes = ramp + [_BODY] * (fill // _BODY) + ramp[::-1]
    assert sum(sizes) == m
    offs, o = [], 0
    for s in sizes:
        offs.append(o)
        o += s
    return sizes, offs


def _proj_kernel(x_hbm, w_ref, b_ref, o_hbm, x_buf, y_buf, in_sems, out_sems):
    sizes, offs = _schedule(x_hbm.shape[0])
    n_chunks = len(sizes)
    wb = w_ref[...].astype(jnp.bfloat16)
    bias = b_ref[...]

    def in_copy(i, j):
        return pltpu.make_async_copy(
            x_hbm.at[pl.ds(offs[i], sizes[i]), :],
            x_buf.at[j, pl.ds(0, sizes[i]), :],
            in_sems.at[j],
        )

    def out_copy(i, j):
        return pltpu.make_async_copy(
            y_buf.at[j, :, pl.ds(0, sizes[i])],
            o_hbm.at[:, pl.ds(offs[i], sizes[i])],
            out_sems.at[j],
        )

    for j in range(min(_NBUF, n_chunks)):
        in_copy(j, j).start()

    for i in range(n_chunks):
        j = i % _NBUF
        in_copy(i, j).wait()
        if i >= _NBUF:
            out_copy(i - _NBUF, j).wait()
        xb = x_buf[j, pl.ds(0, sizes[i]), :].astype(jnp.bfloat16)
        y_buf[j, :, pl.ds(0, sizes[i])] = (
            jax.lax.dot_general(
                wb, xb, (((0,), (1,)), ((), ())),
                preferred_element_type=jnp.float32,
            )
            + bias
        )
        out_copy(i, j).start()
        if i + _NBUF < n_chunks:
            in_copy(i + _NBUF, j).start()

    for i in range(max(0, n_chunks - _NBUF), n_chunks):
        out_copy(i, i % _NBUF).wait()


@jax.jit
def kernel(x, W, b):
    M, K = x.shape
    N = W.shape[1]
    b2 = b.reshape(N, 1)
    yt = pl.pallas_call(
        _proj_kernel,
        in_specs=[
            pl.BlockSpec(memory_space=pltpu.MemorySpace.HBM),
            pl.BlockSpec(memory_space=pltpu.MemorySpace.VMEM),
            pl.BlockSpec(memory_space=pltpu.MemorySpace.VMEM),
        ],
        out_specs=pl.BlockSpec(memory_space=pltpu.MemorySpace.HBM),
        out_shape=jax.ShapeDtypeStruct((N, M), jnp.float32),
        scratch_shapes=[
            pltpu.VMEM((_NBUF, _BODY, K), jnp.float32),
            pltpu.VMEM((_NBUF, N, _BODY), jnp.float32),
            pltpu.SemaphoreType.DMA((_NBUF,)),
            pltpu.SemaphoreType.DMA((_NBUF,)),
        ],
    )(x, W, b2)
    return yt.reshape(_HEADS, N // _HEADS, M).transpose(2, 0, 1)
